# baseline (speedup 1.0000x reference)
import functools

import jax
import jax.numpy as jnp
from jax import lax
from jax.experimental import pallas as pl
from jax.experimental.pallas import tpu as pltpu
from jax.experimental.pallas import tpu_sc as plsc

_CHAR_SIZE = 128
_UNK_IDX = 0
_TABLE_ROWS = 136
_CHUNK = 400

_info = plsc.get_sparse_core_info()
_NC, _NS = _info.num_cores, _info.num_subcores
_NW = _NC * _NS


@functools.lru_cache(maxsize=None)
def _sc_onehot(total):
    per_w = total // _NW
    n_chunks = per_w // _CHUNK
    mesh = plsc.VectorSubcoreMesh(core_axis_name="c", subcore_axis_name="s")

    @functools.partial(
        pl.kernel,
        mesh=mesh,
        out_type=jax.ShapeDtypeStruct((total, _CHAR_SIZE), jnp.int32),
        scratch_types=[
            pltpu.VMEM((_CHUNK,), jnp.int32),
            pltpu.VMEM((_CHUNK, _CHAR_SIZE), jnp.int32),
            pltpu.SemaphoreType.DMA,
        ],
    )
    def k(table_hbm, idx_hbm, out_hbm, idx_v, rows_v, sem):
        wid = lax.axis_index("s") * _NC + lax.axis_index("c")
        base = wid * per_w

        def body(ci, carry):
            off = base + ci * _CHUNK
            pltpu.sync_copy(idx_hbm.at[pl.ds(off, _CHUNK)], idx_v)
            pltpu.async_copy(table_hbm.at[idx_v], rows_v, sem).wait()
            pltpu.sync_copy(rows_v, out_hbm.at[pl.ds(off, _CHUNK)])
            return carry

        lax.fori_loop(0, n_chunks, body, 0)

    return k


def kernel(x):
    n, c = x.shape
    total = n * c
    xf = x.reshape(total)
    pos = lax.iota(jnp.int32, total)
    idx = jnp.where(pos < c * (_UNK_IDX + 1), jnp.int32(_CHAR_SIZE), xf)
    if _UNK_IDX != 0:
        idx = jnp.where(pos < c * _UNK_IDX, xf, idx)
    table = (
        lax.broadcasted_iota(jnp.int32, (_TABLE_ROWS, _CHAR_SIZE), 0)
        == lax.broadcasted_iota(jnp.int32, (_TABLE_ROWS, _CHAR_SIZE), 1)
    ).astype(jnp.int32)
    out = _sc_onehot(total)(table, idx)
    return out.reshape(n, c, _CHAR_SIZE)


# speedup vs baseline: 1.5976x; 1.5976x over previous
import functools

import jax
import jax.numpy as jnp
from jax import lax
from jax.experimental import pallas as pl
from jax.experimental.pallas import tpu as pltpu
from jax.experimental.pallas import tpu_sc as plsc

_CHAR_SIZE = 128
_UNK_IDX = 0
_TABLE_ROWS = 136
_CHUNK = 400
_SC_ROWS = 512
_ROWS_PER_BLOCK = 128

_info = plsc.get_sparse_core_info()
_NC, _NS = _info.num_cores, _info.num_subcores
_NW = _NC * _NS


@functools.lru_cache(maxsize=None)
def _sc_onehot(total):
    per_w = total // _NW
    n_chunks = per_w // _CHUNK
    mesh = plsc.VectorSubcoreMesh(core_axis_name="c", subcore_axis_name="s")

    @functools.partial(
        pl.kernel,
        mesh=mesh,
        out_type=jax.ShapeDtypeStruct((total, _CHAR_SIZE), jnp.int32),
        scratch_types=[
            pltpu.VMEM((_CHUNK,), jnp.int32),
            pltpu.VMEM((_CHUNK, _CHAR_SIZE), jnp.int32),
            pltpu.SemaphoreType.DMA,
        ],
    )
    def k(table_hbm, idx_hbm, out_hbm, idx_v, rows_v, sem):
        wid = lax.axis_index("s") * _NC + lax.axis_index("c")
        base = wid * per_w

        def body(ci, carry):
            off = base + ci * _CHUNK
            pltpu.sync_copy(idx_hbm.at[pl.ds(off, _CHUNK)], idx_v)
            pltpu.async_copy(table_hbm.at[idx_v], rows_v, sem).wait()
            pltpu.sync_copy(rows_v, out_hbm.at[pl.ds(off, _CHUNK)])
            return carry

        lax.fori_loop(0, n_chunks, body, 0)

    return k


def _tc_block(x_ref, o_ref):
    x = x_ref[...]
    r, c = x.shape
    lane = jax.lax.broadcasted_iota(jnp.int32, (r, c, _CHAR_SIZE), 2)
    o_ref[...] = (x[:, :, None] == lane).astype(jnp.int32)


def kernel(x):
    n, c = x.shape
    sc_total = _SC_ROWS * c
    xf = x.reshape(n * c)[:sc_total]
    pos = lax.iota(jnp.int32, sc_total)
    idx = jnp.where(pos < c * (_UNK_IDX + 1), jnp.int32(_CHAR_SIZE), xf)
    if _UNK_IDX != 0:
        idx = jnp.where(pos < c * _UNK_IDX, xf, idx)
    table = (
        lax.broadcasted_iota(jnp.int32, (_TABLE_ROWS, _CHAR_SIZE), 0)
        == lax.broadcasted_iota(jnp.int32, (_TABLE_ROWS, _CHAR_SIZE), 1)
    ).astype(jnp.int32)
    sc_out = _sc_onehot(sc_total)(table, idx)

    tc_rows = n - _SC_ROWS
    tc_out = pl.pallas_call(
        _tc_block,
        grid=(tc_rows // _ROWS_PER_BLOCK,),
        in_specs=[
            pl.BlockSpec(
                (_ROWS_PER_BLOCK, c),
                lambda i: (i + _SC_ROWS // _ROWS_PER_BLOCK, 0),
            )
        ],
        out_specs=pl.BlockSpec(
            (_ROWS_PER_BLOCK, c, _CHAR_SIZE), lambda i: (i, 0, 0)
        ),
        out_shape=jax.ShapeDtypeStruct((tc_rows, c, _CHAR_SIZE), jnp.int32),
    )(x)

    return jnp.concatenate(
        [sc_out.reshape(_SC_ROWS, c, _CHAR_SIZE), tc_out], axis=0
    )


# speedup vs baseline: 5.5434x; 3.4698x over previous
import jax
import jax.numpy as jnp
from jax.experimental import pallas as pl
from jax.experimental.pallas import tpu as pltpu

_CHAR_SIZE = 128
_UNK_IDX = 0
_ROWS_PER_BLOCK = 128


def _onehot_block(x_ref, o_ref):
    i = pl.program_id(0)
    x = x_ref[...]
    r, c = x.shape
    lane = jax.lax.broadcasted_iota(jnp.int32, (r, c, _CHAR_SIZE), 2)
    o_ref[...] = (x[:, :, None] == lane).astype(jnp.int32)

    @pl.when(i == _UNK_IDX // _ROWS_PER_BLOCK)
    def _():
        o_ref[_UNK_IDX % _ROWS_PER_BLOCK] = jnp.zeros(
            (c, _CHAR_SIZE), jnp.int32
        )


def kernel(x):
    n, c = x.shape
    grid = (n // _ROWS_PER_BLOCK,)
    return pl.pallas_call(
        _onehot_block,
        grid=grid,
        in_specs=[pl.BlockSpec((_ROWS_PER_BLOCK, c), lambda i: (i, 0))],
        out_specs=pl.BlockSpec(
            (_ROWS_PER_BLOCK, c, _CHAR_SIZE), lambda i: (i, 0, 0)
        ),
        out_shape=jax.ShapeDtypeStruct((n, c, _CHAR_SIZE), jnp.int32),
        compiler_params=pltpu.CompilerParams(
            dimension_semantics=("parallel",)
        ),
    )(x)


# speedup vs baseline: 5.8199x; 1.0499x over previous
import jax
import jax.numpy as jnp
from jax.experimental import pallas as pl
from jax.experimental.pallas import tpu as pltpu

_CHAR_SIZE = 128
_UNK_IDX = 0
_ROWS_PER_BLOCK = 128
_SUB = 32
_NSUB = _ROWS_PER_BLOCK // _SUB


def _onehot_block(x_ref, o_ref, buf, sems):
    i = pl.program_id(0)
    nsteps = pl.num_programs(0)
    c = x_ref.shape[1]

    for s in range(_NSUB):
        @pl.when(i > 0)
        def _():
            pltpu.make_async_copy(
                buf.at[s],
                o_ref.at[pl.ds((i - 1) * _ROWS_PER_BLOCK + s * _SUB, _SUB)],
                sems.at[s],
            ).wait()

        x = x_ref[pl.ds(s * _SUB, _SUB), :]
        lane = jax.lax.broadcasted_iota(
            jnp.int32, (_SUB, c, _CHAR_SIZE), 2
        )
        oh = (x[:, :, None] == lane).astype(jnp.int32)
        if s == _UNK_IDX // _SUB:
            @pl.when(i == _UNK_IDX // _ROWS_PER_BLOCK)
            def _():
                row = jax.lax.broadcasted_iota(
                    jnp.int32, (_SUB, c, _CHAR_SIZE), 0
                )
                buf[s] = jnp.where(row == _UNK_IDX % _SUB, 0, oh)

            @pl.when(i != _UNK_IDX // _ROWS_PER_BLOCK)
            def _():
                buf[s] = oh
        else:
            buf[s] = oh

        pltpu.make_async_copy(
            buf.at[s],
            o_ref.at[pl.ds(i * _ROWS_PER_BLOCK + s * _SUB, _SUB)],
            sems.at[s],
        ).start()

    @pl.when(i == nsteps - 1)
    def _():
        for s in range(_NSUB):
            pltpu.make_async_copy(
                buf.at[s],
                o_ref.at[pl.ds(i * _ROWS_PER_BLOCK + s * _SUB, _SUB)],
                sems.at[s],
            ).wait()


def kernel(x):
    n, c = x.shape
    grid = (n // _ROWS_PER_BLOCK,)
    return pl.pallas_call(
        _onehot_block,
        grid=grid,
        in_specs=[pl.BlockSpec((_ROWS_PER_BLOCK, c), lambda i: (i, 0))],
        out_specs=pl.BlockSpec(memory_space=pltpu.MemorySpace.HBM),
        out_shape=jax.ShapeDtypeStruct((n, c, _CHAR_SIZE), jnp.int32),
        scratch_shapes=[
            pltpu.VMEM((_NSUB, _SUB, c, _CHAR_SIZE), jnp.int32),
            pltpu.SemaphoreType.DMA((_NSUB,)),
        ],
    )(x)
